# baseline (speedup 1.0000x reference)
import functools

import jax
import jax.numpy as jnp
from jax import lax
from jax.experimental import pallas as pl
from jax.experimental.pallas import tpu as pltpu
from jax.experimental.pallas import tpu_sc as plsc

V = 1000000
D = 64
B = 4096
L = 10
NEG = 5

NC, NS = 2, 16
NW = NC * NS
CHUNKS = 2 + NEG
SEGS = B * CHUNKS
C = B // NW
IDX_W = 128
IDX_ROWS = C * L // IDX_W
LANES = 16


def _sc_sums(idx, u_emb, v_emb):
    mesh = plsc.VectorSubcoreMesh(core_axis_name="c", subcore_axis_name="s")

    @functools.partial(
        pl.kernel,
        mesh=mesh,
        compiler_params=pltpu.CompilerParams(use_tc_tiling_on_sc=False),
        out_type=(
            jax.ShapeDtypeStruct((B, D), jnp.float32),
            jax.ShapeDtypeStruct((B, D), jnp.float32),
            jax.ShapeDtypeStruct((NEG * B, D), jnp.float32),
        ),
        scratch_types=[
            pltpu.VMEM((C * L,), jnp.int32),
            pltpu.VMEM((C * L, D), jnp.float32),
            pltpu.VMEM((C, D), jnp.float32),
            pltpu.SemaphoreType.DMA,
        ],
    )
    def k(idx_hbm, u_hbm, v_hbm, out_u, out_v, out_n, idx_v, rows_v, sum_v, sem):
        wid = lax.axis_index("s") * NC + lax.axis_index("c")
        for ck in range(CHUNKS):
            base = (ck * B + wid * C) * L
            pltpu.sync_copy(idx_hbm.at[pl.ds(base, C * L)], idx_v)
            tbl = u_hbm if ck == 0 else v_hbm
            copies = [
                pltpu.async_copy(
                    tbl.at[idx_v.at[pl.ds(j * IDX_W, IDX_W)]],
                    rows_v.at[pl.ds(j * IDX_W, IDX_W)],
                    sem,
                )
                for j in range(IDX_ROWS)
            ]
            for c in copies:
                c.wait()

            def body(b, carry):
                base = b * L
                for dblk in range(D // LANES):
                    sl = pl.ds(dblk * LANES, LANES)
                    acc = rows_v[base, sl]
                    for l in range(1, L):
                        acc = acc + rows_v[base + l, sl]
                    sum_v[b, sl] = acc
                return carry

            lax.fori_loop(0, C, body, 0)

            if ck == 0:
                dst = out_u.at[pl.ds(wid * C, C)]
            elif ck == 1:
                dst = out_v.at[pl.ds(wid * C, C)]
            else:
                dst = out_n.at[pl.ds((ck - 2) * B + wid * C, C)]
            pltpu.sync_copy(sum_v, dst)

    return k(idx, u_emb, v_emb)


def _log_sigmoid(x):
    return jnp.minimum(x, 0.0) - jnp.log1p(jnp.exp(-jnp.abs(x)))


def _tc_loss(su, sv, sn):
    def body(su_ref, sv_ref, sn_ref, out_ref):
        u = su_ref[...]
        inv = 1.0 / float(L * L)
        score = jnp.sum(u * sv_ref[...], axis=1, keepdims=True) * inv
        acc = _log_sigmoid(score)
        for j in range(NEG):
            nsc = jnp.sum(sn_ref[pl.ds(j * B, B), :] * u, axis=1, keepdims=True) * inv
            acc = acc + _log_sigmoid(-nsc)
        out_ref[...] = jnp.reshape(-jnp.sum(acc) / float(B), (1, 1))

    return pl.pallas_call(
        body,
        out_shape=jax.ShapeDtypeStruct((1, 1), jnp.float32),
    )(su, sv, sn)


def kernel(pos_u, pos_v, neg_v, u_emb, v_emb):
    negp = neg_v.reshape(B, NEG, L).transpose(1, 0, 2)
    idx = jnp.concatenate(
        [pos_u.reshape(-1), pos_v.reshape(-1), negp.reshape(-1)]
    ).astype(jnp.int32)
    su, sv, sn = _sc_sums(idx, u_emb, v_emb)
    return _tc_loss(su, sv, sn)[0, 0]


# speedup vs baseline: 1.1580x; 1.1580x over previous
import functools

import jax
import jax.numpy as jnp
from jax import lax
from jax.experimental import pallas as pl
from jax.experimental.pallas import tpu as pltpu
from jax.experimental.pallas import tpu_sc as plsc

V = 1000000
D = 64
B = 4096
L = 10
NEG = 5

NC, NS = 2, 16
NW = NC * NS
CHUNKS = 2 + NEG
SEGS = B * CHUNKS
C = 64
SUB = B // NW // C
IDX_W = 128
IDX_ROWS = C * L // IDX_W
LANES = 16


def _sc_sums(idx, z):
    mesh = plsc.VectorSubcoreMesh(core_axis_name="c", subcore_axis_name="s")

    @functools.partial(
        pl.kernel,
        mesh=mesh,
        out_type=(
            jax.ShapeDtypeStruct((B, D), jnp.float32),
            jax.ShapeDtypeStruct((B, D), jnp.float32),
            jax.ShapeDtypeStruct((NEG * B, D), jnp.float32),
        ),
        scratch_types=[
            pltpu.VMEM((C * L,), jnp.int32),
            pltpu.VMEM((C * L, 2 * D), jnp.float32),
            pltpu.VMEM((C, D), jnp.float32),
            pltpu.SemaphoreType.DMA,
        ],
    )
    def k(idx_hbm, z_hbm, out_u, out_v, out_n, idx_v, rows_v, sum_v, sem):
        wid = lax.axis_index("s") * NC + lax.axis_index("c")
        for ck in range(CHUNKS):
            lo = 0 if ck == 0 else D
            for h in range(SUB):
                seg0 = ck * B + wid * (C * SUB) + h * C
                pltpu.sync_copy(idx_hbm.at[pl.ds(seg0 * L, C * L)], idx_v)
                copies = [
                    pltpu.async_copy(
                        z_hbm.at[idx_v.at[pl.ds(j * IDX_W, IDX_W)]],
                        rows_v.at[pl.ds(j * IDX_W, IDX_W)],
                        sem,
                    )
                    for j in range(IDX_ROWS)
                ]
                for c in copies:
                    c.wait()

                def body(b, carry):
                    base = b * L
                    for dblk in range(D // LANES):
                        sl = pl.ds(lo + dblk * LANES, LANES)
                        acc = rows_v[base, sl]
                        for l in range(1, L):
                            acc = acc + rows_v[base + l, sl]
                        sum_v[b, pl.ds(dblk * LANES, LANES)] = acc
                    return carry

                lax.fori_loop(0, C, body, 0)

                if ck == 0:
                    dst = out_u.at[pl.ds(seg0, C)]
                elif ck == 1:
                    dst = out_v.at[pl.ds(seg0 - B, C)]
                else:
                    dst = out_n.at[pl.ds(seg0 - 2 * B, C)]
                pltpu.sync_copy(sum_v, dst)

    return k(idx, z)


def _log_sigmoid(x):
    return jnp.minimum(x, 0.0) - jnp.log1p(jnp.exp(-jnp.abs(x)))


def _tc_loss(su, sv, sn):
    def body(su_ref, sv_ref, sn_ref, out_ref):
        u = su_ref[...]
        inv = 1.0 / float(L * L)
        score = jnp.sum(u * sv_ref[...], axis=1, keepdims=True) * inv
        acc = _log_sigmoid(score)
        for j in range(NEG):
            nsc = jnp.sum(sn_ref[pl.ds(j * B, B), :] * u, axis=1, keepdims=True) * inv
            acc = acc + _log_sigmoid(-nsc)
        out_ref[...] = jnp.reshape(-jnp.sum(acc) / float(B), (1, 1))

    return pl.pallas_call(
        body,
        out_shape=jax.ShapeDtypeStruct((1, 1), jnp.float32),
    )(su, sv, sn)


def kernel(pos_u, pos_v, neg_v, u_emb, v_emb):
    z = jnp.concatenate([u_emb, v_emb], axis=1)
    negp = neg_v.reshape(B, NEG, L).transpose(1, 0, 2)
    idx = jnp.concatenate(
        [pos_u.reshape(-1), pos_v.reshape(-1), negp.reshape(-1)]
    ).astype(jnp.int32)
    su, sv, sn = _sc_sums(idx, z)
    return _tc_loss(su, sv, sn)[0, 0]


# speedup vs baseline: 1.9019x; 1.6423x over previous
import functools

import jax
import jax.numpy as jnp
from jax import lax
from jax.experimental import pallas as pl
from jax.experimental.pallas import tpu as pltpu
from jax.experimental.pallas import tpu_sc as plsc

V = 1000000
D = 64
B = 4096
L = 10
NEG = 5

NC, NS = 2, 16
NW = NC * NS
CHUNKS = 2 + NEG
SEGS = B * CHUNKS
C = 64
SUB = B // NW // C
IDX_W = 128
IDX_ROWS = C * L // IDX_W
LANES = 16


def _sc_sums(idx, z):
    mesh = plsc.VectorSubcoreMesh(core_axis_name="c", subcore_axis_name="s")

    @functools.partial(
        pl.kernel,
        mesh=mesh,
        out_type=(
            jax.ShapeDtypeStruct((B, D), jnp.float32),
            jax.ShapeDtypeStruct((B, D), jnp.float32),
            jax.ShapeDtypeStruct((NEG * B, D), jnp.float32),
        ),
        scratch_types=[
            pltpu.VMEM((C * L,), jnp.int32),
            pltpu.VMEM((C * L, 2 * D), jnp.float32),
            pltpu.VMEM((C, D), jnp.float32),
            pltpu.SemaphoreType.DMA,
        ],
    )
    def k(idx_hbm, z_hbm, out_u, out_v, out_n, idx_v, rows_v, sum_v, sem):
        wid = lax.axis_index("s") * NC + lax.axis_index("c")
        for ck in range(CHUNKS):
            lo = 0 if ck == 0 else D
            for h in range(SUB):
                seg0 = ck * B + wid * (C * SUB) + h * C
                pltpu.sync_copy(idx_hbm.at[pl.ds(seg0 * L, C * L)], idx_v)
                copies = [
                    pltpu.async_copy(
                        z_hbm.at[idx_v.at[pl.ds(j * IDX_W, IDX_W)]],
                        rows_v.at[pl.ds(j * IDX_W, IDX_W)],
                        sem,
                    )
                    for j in range(IDX_ROWS)
                ]
                for c in copies:
                    c.wait()

                def body(b, carry):
                    base = b * L
                    for dblk in range(D // LANES):
                        sl = pl.ds(lo + dblk * LANES, LANES)
                        acc = rows_v[base, sl]
                        for l in range(1, L):
                            acc = acc + rows_v[base + l, sl]
                        sum_v[b, pl.ds(dblk * LANES, LANES)] = acc
                    return carry

                lax.fori_loop(0, C, body, 0)

                if ck == 0:
                    dst = out_u.at[pl.ds(seg0, C)]
                elif ck == 1:
                    dst = out_v.at[pl.ds(seg0 - B, C)]
                else:
                    dst = out_n.at[pl.ds(seg0 - 2 * B, C)]
                pltpu.sync_copy(sum_v, dst)

    return k(idx, z)


PCH = 16384


def _pack_tables(ut, vt):
    def body(u_ref, v_ref, z_ref):
        eye = (
            lax.broadcasted_iota(jnp.int32, (D, D), 0)
            == lax.broadcasted_iota(jnp.int32, (D, D), 1)
        ).astype(jnp.float32)
        dn = (((0,), (0,)), ((), ()))
        xtu = lax.dot_general(u_ref[...], eye, dn, preferred_element_type=jnp.float32)
        xtv = lax.dot_general(v_ref[...], eye, dn, preferred_element_type=jnp.float32)
        z_ref[...] = jnp.concatenate([xtu, xtv], axis=1)

    grid = (V + PCH - 1) // PCH
    return pl.pallas_call(
        body,
        grid=(grid,),
        in_specs=[
            pl.BlockSpec((D, PCH), lambda i: (0, i)),
            pl.BlockSpec((D, PCH), lambda i: (0, i)),
        ],
        out_specs=pl.BlockSpec((PCH, 2 * D), lambda i: (i, 0)),
        out_shape=jax.ShapeDtypeStruct((V, 2 * D), jnp.float32),
    )(ut, vt)


def _log_sigmoid(x):
    return jnp.minimum(x, 0.0) - jnp.log1p(jnp.exp(-jnp.abs(x)))


def _tc_loss(su, sv, sn):
    def body(su_ref, sv_ref, sn_ref, out_ref):
        u = su_ref[...]
        inv = 1.0 / float(L * L)
        score = jnp.sum(u * sv_ref[...], axis=1, keepdims=True) * inv
        acc = _log_sigmoid(score)
        for j in range(NEG):
            nsc = jnp.sum(sn_ref[pl.ds(j * B, B), :] * u, axis=1, keepdims=True) * inv
            acc = acc + _log_sigmoid(-nsc)
        out_ref[...] = jnp.reshape(-jnp.sum(acc) / float(B), (1, 1))

    return pl.pallas_call(
        body,
        out_shape=jax.ShapeDtypeStruct((1, 1), jnp.float32),
    )(su, sv, sn)


def kernel(pos_u, pos_v, neg_v, u_emb, v_emb):
    z = _pack_tables(u_emb.T, v_emb.T)
    negp = neg_v.reshape(B, NEG, L).transpose(1, 0, 2)
    idx = jnp.concatenate(
        [pos_u.reshape(-1), pos_v.reshape(-1), negp.reshape(-1)]
    ).astype(jnp.int32)
    su, sv, sn = _sc_sums(idx, z)
    return _tc_loss(su, sv, sn)[0, 0]


# speedup vs baseline: 2.2572x; 1.1868x over previous
import functools

import jax
import jax.numpy as jnp
from jax import lax
from jax.experimental import pallas as pl
from jax.experimental.pallas import tpu as pltpu
from jax.experimental.pallas import tpu_sc as plsc

V = 1000000
D = 64
B = 4096
L = 10
NEG = 5

NC, NS = 2, 16
NW = NC * NS
CHUNKS = 2 + NEG
G = B // NW
GH = G // 2
LANES = 16


def _sc_sums(put, pvt, nvt, z):
    mesh = plsc.VectorSubcoreMesh(core_axis_name="c", subcore_axis_name="s")

    @functools.partial(
        pl.kernel,
        mesh=mesh,
        out_type=(
            jax.ShapeDtypeStruct((B, D), jnp.float32),
            jax.ShapeDtypeStruct((B, D), jnp.float32),
            jax.ShapeDtypeStruct((NEG * B, D), jnp.float32),
        ),
        scratch_types=[
            pltpu.VMEM((L, G), jnp.int32),
            pltpu.VMEM((L * GH, 2 * D), jnp.float32),
            pltpu.VMEM((G, D), jnp.float32),
            pltpu.SemaphoreType.DMA,
        ],
    )
    def k(put_h, pvt_h, nvt_h, z_hbm, out_u, out_v, out_n, idx2, rows_v, sum_v, sem):
        wid = lax.axis_index("s") * NC + lax.axis_index("c")
        for ck in range(CHUNKS):
            if ck == 0:
                src, c0, lo = put_h, wid * G, 0
            elif ck == 1:
                src, c0, lo = pvt_h, wid * G, D
            else:
                src, c0, lo = nvt_h, (ck - 2) * B + wid * G, D
            pltpu.sync_copy(src.at[:, pl.ds(c0, G)], idx2)

            for h in range(2):
                copies = [
                    pltpu.async_copy(
                        z_hbm.at[idx2.at[l, pl.ds(h * GH, GH)]],
                        rows_v.at[pl.ds(l * GH, GH)],
                        sem,
                    )
                    for l in range(L)
                ]
                for c in copies:
                    c.wait()

                def body(s, carry):
                    for dblk in range(D // LANES):
                        sl = pl.ds(lo + dblk * LANES, LANES)
                        acc = rows_v[s, sl]
                        for l in range(1, L):
                            acc = acc + rows_v[l * GH + s, sl]
                        sum_v[h * GH + s, pl.ds(dblk * LANES, LANES)] = acc
                    return carry

                lax.fori_loop(0, GH, body, 0)

            if ck == 0:
                dst = out_u.at[pl.ds(wid * G, G)]
            elif ck == 1:
                dst = out_v.at[pl.ds(wid * G, G)]
            else:
                dst = out_n.at[pl.ds((ck - 2) * B + wid * G, G)]
            pltpu.sync_copy(sum_v, dst)

    return k(put, pvt, nvt, z)


PCH = 16384


def _pack_tables(ut, vt):
    def body(u_ref, v_ref, z_ref):
        eye = (
            lax.broadcasted_iota(jnp.int32, (D, D), 0)
            == lax.broadcasted_iota(jnp.int32, (D, D), 1)
        ).astype(jnp.float32)
        dn = (((0,), (0,)), ((), ()))
        xtu = lax.dot_general(u_ref[...], eye, dn, preferred_element_type=jnp.float32)
        xtv = lax.dot_general(v_ref[...], eye, dn, preferred_element_type=jnp.float32)
        z_ref[...] = jnp.concatenate([xtu, xtv], axis=1)

    grid = (V + PCH - 1) // PCH
    return pl.pallas_call(
        body,
        grid=(grid,),
        in_specs=[
            pl.BlockSpec((D, PCH), lambda i: (0, i)),
            pl.BlockSpec((D, PCH), lambda i: (0, i)),
        ],
        out_specs=pl.BlockSpec((PCH, 2 * D), lambda i: (i, 0)),
        out_shape=jax.ShapeDtypeStruct((V, 2 * D), jnp.float32),
    )(ut, vt)


def _log_sigmoid(x):
    return jnp.minimum(x, 0.0) - jnp.log1p(jnp.exp(-jnp.abs(x)))


def _tc_loss(su, sv, sn):
    def body(su_ref, sv_ref, sn_ref, out_ref):
        u = su_ref[...]
        inv = 1.0 / float(L * L)
        score = jnp.sum(u * sv_ref[...], axis=1, keepdims=True) * inv
        acc = jnp.sum(_log_sigmoid(score))
        urep = jnp.reshape(
            jnp.broadcast_to(u[:, None, :], (B, NEG, D)), (B * NEG, D)
        )
        nsc = jnp.sum(sn_ref[...] * urep, axis=1, keepdims=True) * inv
        acc = acc + jnp.sum(_log_sigmoid(-nsc))
        out_ref[...] = jnp.reshape(-acc / float(B), (1, 1))

    return pl.pallas_call(
        body,
        out_shape=jax.ShapeDtypeStruct((1, 1), jnp.float32),
    )(su, sv, sn)


def kernel(pos_u, pos_v, neg_v, u_emb, v_emb):
    z = _pack_tables(u_emb.T, v_emb.T)
    su, sv, sn = _sc_sums(
        pos_u.T.astype(jnp.int32),
        pos_v.T.astype(jnp.int32),
        neg_v.T.astype(jnp.int32),
        z,
    )
    return _tc_loss(su, sv, sn)[0, 0]


# speedup vs baseline: 2.5970x; 1.1505x over previous
import functools

import jax
import jax.numpy as jnp
from jax import lax
from jax.experimental import pallas as pl
from jax.experimental.pallas import tpu as pltpu
from jax.experimental.pallas import tpu_sc as plsc

V = 1000000
D = 64
B = 4096
L = 10
NEG = 5

NC, NS = 2, 16
NW = NC * NS
G = B // NW
GN = NEG * G
GH = 40
LANES = 16


def _sc_sums(put, pvt, nvt, z):
    mesh = plsc.VectorSubcoreMesh(core_axis_name="c", subcore_axis_name="s")

    @functools.partial(
        pl.kernel,
        mesh=mesh,
        out_type=(
            jax.ShapeDtypeStruct((B, D), jnp.float32),
            jax.ShapeDtypeStruct((B, D), jnp.float32),
            jax.ShapeDtypeStruct((NEG * B, D), jnp.float32),
        ),
        scratch_types=[
            pltpu.VMEM((L, GN), jnp.int32),
            pltpu.VMEM((L * GN,), jnp.int32),
            pltpu.VMEM((L * GH, 2 * D), jnp.float32),
            pltpu.VMEM((GN // 2, D), jnp.float32),
            pltpu.SemaphoreType.DMA,
        ],
    )
    def k(put_h, pvt_h, nvt_h, z_hbm, out_u, out_v, out_n,
          idx2, idx1, rows_v, sum_v, sem):
        wid = lax.axis_index("s") * NC + lax.axis_index("c")
        for ck in range(3):
            if ck == 0:
                src, c0, lo, W = put_h, wid * G, 0, G
            elif ck == 1:
                src, c0, lo, W = pvt_h, wid * G, D, G
            else:
                src, c0, lo, W = nvt_h, wid * GN, D, GN
            pltpu.sync_copy(src.at[:, pl.ds(c0, W)], idx2.at[:, pl.ds(0, W)])

            def rl_body(c, carry):
                cb = pl.multiple_of(c * LANES, LANES)
                for l in range(L):
                    idx1[pl.ds(l * GN + cb, LANES)] = idx2[l, pl.ds(cb, LANES)]
                return carry

            lax.fori_loop(0, W // LANES, rl_body, 0)

            def fire(base, n_seg):
                return [
                    pltpu.async_copy(
                        z_hbm.at[idx1.at[pl.ds(l * GN + base, n_seg)]],
                        rows_v.at[pl.ds(l * GH, n_seg)],
                        sem,
                    )
                    for l in range(L)
                ]

            def accum(s, dest):
                for dblk in range(D // LANES):
                    sl = pl.ds(lo + dblk * LANES, LANES)
                    acc = rows_v[s, sl]
                    for l in range(1, L):
                        acc = acc + rows_v[l * GH + s, sl]
                    sum_v[dest, pl.ds(dblk * LANES, LANES)] = acc

            if ck < 2:
                def hbody(h, carry):
                    base = pl.multiple_of(h * 32, 32)
                    copies = fire(base, 32)
                    for c in copies:
                        c.wait()

                    def body(s, carry2):
                        accum(s, h * 32 + s)
                        return carry2

                    lax.fori_loop(0, 32, body, 0)
                    return carry

                lax.fori_loop(0, G // 32, hbody, 0)
                dst = out_u if ck == 0 else out_v
                pltpu.sync_copy(sum_v.at[pl.ds(0, G)], dst.at[pl.ds(wid * G, G)])
            else:
                GB = G // 2
                for half in range(2):
                    def hbody(h, carry):
                        base = pl.multiple_of(half * (GN // 2) + h * GH, 8)
                        copies = fire(base, GH)
                        for c in copies:
                            c.wait()

                        def body(bq, carry2):
                            for n in range(NEG):
                                accum(bq * NEG + n, n * GB + h * 8 + bq)
                            return carry2

                        lax.fori_loop(0, 8, body, 0)
                        return carry

                    lax.fori_loop(0, GB // 8, hbody, 0)
                    for n in range(NEG):
                        pltpu.sync_copy(
                            sum_v.at[pl.ds(n * GB, GB)],
                            out_n.at[pl.ds(n * B + wid * G + half * GB, GB)],
                        )

    return k(put, pvt, nvt, z)


PCH = 16384


def _pack_tables(ut, vt):
    def body(u_ref, v_ref, z_ref):
        eye = (
            lax.broadcasted_iota(jnp.int32, (2 * D, 2 * D), 0)
            == lax.broadcasted_iota(jnp.int32, (2 * D, 2 * D), 1)
        ).astype(jnp.float32)
        dn = (((0,), (0,)), ((), ()))
        x = jnp.concatenate([u_ref[...], v_ref[...]], axis=0)
        z_ref[...] = lax.dot_general(x, eye, dn, preferred_element_type=jnp.float32)

    grid = (V + PCH - 1) // PCH
    return pl.pallas_call(
        body,
        grid=(grid,),
        compiler_params=pltpu.CompilerParams(fuse_transposed_lhs_in_matmul=True),
        in_specs=[
            pl.BlockSpec((D, PCH), lambda i: (0, i)),
            pl.BlockSpec((D, PCH), lambda i: (0, i)),
        ],
        out_specs=pl.BlockSpec((PCH, 2 * D), lambda i: (i, 0)),
        out_shape=jax.ShapeDtypeStruct((V, 2 * D), jnp.float32),
    )(ut, vt)


def _log_sigmoid(x):
    return jnp.minimum(x, 0.0) - jnp.log1p(jnp.exp(-jnp.abs(x)))


def _tc_loss(su, sv, sn):
    def body(su_ref, sv_ref, sn_ref, out_ref):
        u = su_ref[...]
        inv = 1.0 / float(L * L)
        ones = jnp.full((D, 1), inv, jnp.float32)
        dn = (((1,), (0,)), ((), ()))
        score = lax.dot_general(
            u * sv_ref[...], ones, dn, preferred_element_type=jnp.float32
        )
        acc = jnp.sum(_log_sigmoid(score))
        for j in range(NEG):
            nsc = lax.dot_general(
                sn_ref[pl.ds(j * B, B), :] * u, ones, dn,
                preferred_element_type=jnp.float32,
            )
            acc = acc + jnp.sum(_log_sigmoid(-nsc))
        out_ref[...] = jnp.reshape(-acc / float(B), (1, 1))

    return pl.pallas_call(
        body,
        out_shape=jax.ShapeDtypeStruct((1, 1), jnp.float32),
    )(su, sv, sn)


def kernel(pos_u, pos_v, neg_v, u_emb, v_emb):
    z = _pack_tables(u_emb.T, v_emb.T)
    su, sv, sn = _sc_sums(
        pos_u.T.astype(jnp.int32),
        pos_v.T.astype(jnp.int32),
        neg_v.T.astype(jnp.int32),
        z,
    )
    return _tc_loss(su, sv, sn)[0, 0]
